# Initial kernel scaffold; baseline (speedup 1.0000x reference)
import jax
import jax.numpy as jnp
from jax.experimental import pallas as pl


def kernel(h, edge_index, e, W_emb, b_emb, W, a, gamma, beta, M0, mb0, M1, mb1, M2, mb2):
    raise NotImplementedError("write your pallas kernel here")


# speedup vs baseline: 18.7014x; 18.7014x over previous
import jax
import jax.numpy as jnp
from jax.experimental import pallas as pl
from jax.experimental.pallas import tpu as pltpu


def _embed_body(h_ref, w_ref, b_ref, o_ref):
    o_ref[...] = (
        jnp.dot(h_ref[...], w_ref[...], preferred_element_type=jnp.float32)
        + b_ref[...]
    )


def kernel(h, edge_index, e, W_emb, b_emb, W, a, gamma, beta, M0, mb0, M1, mb1, M2, mb2):
    out_dtype = jnp.result_type(h.dtype, W_emb.dtype, M2.dtype)
    f32 = jnp.float32
    h, W_emb, b_emb, W, a, gamma, beta = (
        x.astype(f32) for x in (h, W_emb, b_emb, W, a, gamma, beta))
    M0, mb0, M1, mb1, M2, mb2 = (
        x.astype(f32) for x in (M0, mb0, M1, mb1, M2, mb2))
    edge_index = edge_index.astype(jnp.int32)
    with jax.enable_x64(False):
        out = _run(h, edge_index, W_emb, b_emb, W, a, gamma, beta,
                   M0, mb0, M1, mb1, M2, mb2)
    return out.astype(out_dtype)


def _run(h, edge_index, W_emb, b_emb, W, a, gamma, beta, M0, mb0, M1, mb1, M2, mb2):
    N = h.shape[0]
    D = W_emb.shape[1]
    h0 = pl.pallas_call(
        _embed_body,
        out_shape=jax.ShapeDtypeStruct((N, D), jnp.float32),
        grid=(10,),
        in_specs=[
            pl.BlockSpec((N // 10, 128), lambda i: (i, 0)),
            pl.BlockSpec((128, D), lambda i: (0, 0)),
            pl.BlockSpec((1, D), lambda i: (0, 0)),
        ],
        out_specs=pl.BlockSpec((N // 10, D), lambda i: (i, 0)),
    )(h, W_emb, b_emb.reshape(1, -1))

    src = edge_index[0]
    dst = edge_index[1]
    hcur = h0
    n_layers = W.shape[0]
    n_heads = W.shape[1]
    for l in range(n_layers):
        h_in = hcur
        outs = []
        for hd in range(n_heads):
            z = hcur @ W[l, hd]
            z_src = jnp.take(z, src, axis=0)
            z_dst = jnp.take(z, dst, axis=0)
            logits = jnp.concatenate([z_src, z_dst], axis=-1) @ a[l, hd]
            logits = jax.nn.leaky_relu(logits, negative_slope=0.01)
            m = jax.ops.segment_max(logits, dst, num_segments=N)
            m = jnp.where(jnp.isfinite(m), m, 0.0)
            ex = jnp.exp(logits - jnp.take(m, dst))
            denom = jax.ops.segment_sum(ex, dst, num_segments=N)
            alpha = ex / (jnp.take(denom, dst) + 1e-9)
            hn = jax.ops.segment_sum(alpha[:, None] * z_src, dst, num_segments=N)
            hn = hn * gamma[l, hd] + beta[l, hd]
            hn = jax.nn.elu(hn)
            outs.append(hn)
        hcur = h_in + jnp.concatenate(outs, axis=-1)
    hg = jnp.mean(hcur, axis=0, keepdims=True)
    x = jax.nn.relu(hg @ M0 + mb0)
    x = jax.nn.relu(x @ M1 + mb1)
    return x @ M2 + mb2


# speedup vs baseline: 1292.5580x; 69.1157x over previous
import jax
import jax.numpy as jnp
from jax import lax
from jax.experimental import pallas as pl
from jax.experimental.pallas import tpu as pltpu
from jax.experimental.pallas import tpu_sc as plsc

N = 10000
E = 320000
H = 8
DH = 16
D = H * DH
NC = 2
NS = 16
NW = NC * NS
EW = E // NW
CH = 80
NCH = EW // CH
ZW = 144
RB = 1000
NEG = -1e30

f32 = jnp.float32
i32 = jnp.int32


def _prep_outputs(z, i, zr_ref, sd_ref, ds_ref, sm_ref, aa_ref):
    sd2 = jnp.dot(z, aa_ref[...], preferred_element_type=f32)
    sd_ref[...] = sd2[:, 0:16]
    ds_ref[...] = sd2[:, 16:32]
    rb = z.shape[0]
    zr_ref[...] = jnp.concatenate(
        [z, jnp.ones((rb, 8), f32), jnp.zeros((rb, 8), f32)], axis=1)
    bm = jnp.max(sd2[:, 0:16], axis=0, keepdims=True)

    @pl.when(i == 0)
    def _():
        sm_ref[...] = jnp.full((1, 16), NEG, f32)

    sm_ref[...] = jnp.maximum(sm_ref[...], bm)


def _dense0_body(h_ref, we_ref, be_ref, wc_ref, aa_ref,
                 h0_ref, zr_ref, sd_ref, ds_ref, sm_ref):
    i = pl.program_id(0)
    h0 = jnp.dot(h_ref[...], we_ref[...], preferred_element_type=f32) + be_ref[...]
    h0_ref[...] = h0
    z = jnp.dot(h0, wc_ref[...], preferred_element_type=f32)
    _prep_outputs(z, i, zr_ref, sd_ref, ds_ref, sm_ref, aa_ref)


def _post_head(hin_ref, a0_ref, a1_ref, g_ref, b_ref, p_ref):
    acc = a0_ref[...] + a1_ref[...]
    den = jnp.dot(acc[:, 128:136], p_ref[...], preferred_element_type=f32) + 1e-9
    t = acc[:, 0:128] / den * g_ref[...] + b_ref[...]
    hn = jnp.where(t > 0, t, jnp.exp(jnp.minimum(t, 0.0)) - 1.0)
    return hin_ref[...] + hn


def _post_body(hin_ref, a0_ref, a1_ref, g_ref, b_ref, p_ref, wc_ref, aa_ref,
               hn_ref, zr_ref, sd_ref, ds_ref, sm_ref):
    i = pl.program_id(0)
    hv = _post_head(hin_ref, a0_ref, a1_ref, g_ref, b_ref, p_ref)
    hn_ref[...] = hv
    z = jnp.dot(hv, wc_ref[...], preferred_element_type=f32)
    _prep_outputs(z, i, zr_ref, sd_ref, ds_ref, sm_ref, aa_ref)


def _final_body(hin_ref, a0_ref, a1_ref, g_ref, b_ref, p_ref,
                m0_ref, b0_ref, m1_ref, b1_ref, m2_ref, b2_ref,
                out_ref, sacc_ref):
    i = pl.program_id(0)
    hv = _post_head(hin_ref, a0_ref, a1_ref, g_ref, b_ref, p_ref)
    part = jnp.sum(hv, axis=0, keepdims=True)

    @pl.when(i == 0)
    def _():
        sacc_ref[...] = jnp.zeros_like(sacc_ref)

    sacc_ref[...] += part

    @pl.when(i == pl.num_programs(0) - 1)
    def _():
        hg = sacc_ref[...] * (1.0 / N)
        x = jnp.maximum(
            jnp.dot(hg, m0_ref[...], preferred_element_type=f32) + b0_ref[...], 0.0)
        x = jnp.maximum(
            jnp.dot(x, m1_ref[...], preferred_element_type=f32) + b1_ref[...], 0.0)
        out_ref[...] = jnp.dot(x, m2_ref[...], preferred_element_type=f32) + b2_ref[...]


def _row_spec(w):
    return pl.BlockSpec((RB, w), lambda i: (i, 0))


def _full_spec(r, c):
    return pl.BlockSpec((r, c), lambda i: (0, 0))


_GRID = N // RB

_STATE_SHAPES = [jax.ShapeDtypeStruct((N, D), f32),
                 jax.ShapeDtypeStruct((N, ZW), f32),
                 jax.ShapeDtypeStruct((N, 16), f32),
                 jax.ShapeDtypeStruct((N, 16), f32),
                 jax.ShapeDtypeStruct((1, 16), f32)]
_STATE_SPECS = [_row_spec(D), _row_spec(ZW), _row_spec(16), _row_spec(16),
                _full_spec(1, 16)]

_dense0 = pl.pallas_call(
    _dense0_body,
    grid=(_GRID,),
    in_specs=[_row_spec(D), _full_spec(D, D), _full_spec(1, D),
              _full_spec(D, D), _full_spec(D, 32)],
    out_specs=_STATE_SPECS,
    out_shape=_STATE_SHAPES,
)

_post = pl.pallas_call(
    _post_body,
    grid=(_GRID,),
    in_specs=[_row_spec(D), _row_spec(ZW), _row_spec(ZW), _full_spec(1, D),
              _full_spec(1, D), _full_spec(8, D), _full_spec(D, D),
              _full_spec(D, 32)],
    out_specs=_STATE_SPECS,
    out_shape=_STATE_SHAPES,
)

_final = pl.pallas_call(
    _final_body,
    grid=(_GRID,),
    in_specs=[_row_spec(D), _row_spec(ZW), _row_spec(ZW), _full_spec(1, D),
              _full_spec(1, D), _full_spec(8, D), _full_spec(D, 64),
              _full_spec(1, 64), _full_spec(64, 32), _full_spec(1, 32),
              _full_spec(32, 10), _full_spec(1, 10)],
    out_specs=pl.BlockSpec((1, 10), lambda i: (0, 0)),
    out_shape=jax.ShapeDtypeStruct((1, 10), f32),
    scratch_shapes=[pltpu.VMEM((1, D), f32)],
)


_MESH = plsc.VectorSubcoreMesh(core_axis_name="c", subcore_axis_name="s")
_NOTC = pltpu.CompilerParams(use_tc_tiling_on_sc=False)

_GDN = lax.GatherDimensionNumbers(
    offset_dims=(), collapsed_slice_dims=(0,), start_index_map=(0,))


def _vgather(vec, idx):
    return lax.gather(vec, idx[:, None], _GDN, (1,),
                      mode=lax.GatherScatterMode.PROMISE_IN_BOUNDS)


def _edge_body(sd_hbm, ds_hbm, sm_hbm, zr_hbm, src2_hbm, dst2_hbm,
               acc_hbm, zv, av, bv, srcv, dstv, smv, zbuf, acc_sh):
    cc = lax.axis_index("c")
    ss = lax.axis_index("s")
    wid = ss * NC + cc
    iota = lax.iota(i32, 16)
    lane8 = iota < 8
    zero = jnp.zeros((16,), f32)

    pltpu.sync_copy(sm_hbm, smv)

    def zrow(j, carry):
        for c9 in range(ZW // 16):
            zbuf[j, pl.ds(c9 * 16, 16)] = zero
        return carry
    lax.fori_loop(0, 100, zrow, 0)

    def zchunk(k, carry):
        cid = ss + k * NS

        @pl.when(cid < N // 100)
        def _():
            pltpu.sync_copy(zbuf, acc_sh.at[pl.ds(cid * 100, 100), :])
        return carry
    lax.fori_loop(0, (N // 100 + NS - 1) // NS, zchunk, 0)
    plsc.subcore_barrier()

    smaxv = smv[pl.ds(0, 16)]

    def chunk(ci, carry):
        pltpu.sync_copy(src2_hbm.at[wid, ci], srcv)
        pltpu.sync_copy(dst2_hbm.at[wid, ci], dstv)
        pltpu.sync_copy(sd_hbm.at[srcv], av)
        pltpu.sync_copy(ds_hbm.at[dstv], bv)
        pltpu.sync_copy(zr_hbm.at[srcv], zv)

        def edge(k, ecarry):
            a = av[k]
            b = bv[k]
            t = a + b
            lr = jnp.maximum(t, t * 0.01)
            dg = b + smaxv
            m = jnp.maximum(dg, dg * 0.01)
            ex = jnp.exp(lr - m)
            exm = jnp.where(lane8, ex, 0.0)
            for hh in range(H):
                bc = _vgather(exm, jnp.full((16,), hh, i32))
                sl = pl.ds(hh * 16, 16)
                zv[k, sl] = zv[k, sl] * bc
            zv[k, pl.ds(128, 16)] = exm
            return ecarry
        lax.fori_loop(0, CH, edge, 0)
        pltpu.sync_copy(zv, acc_sh.at[dstv], add=True)
        return carry
    lax.fori_loop(0, NCH, chunk, 0)

    plsc.subcore_barrier()

    def ochunk(k, carry):
        cid = ss + k * NS

        @pl.when(cid < N // 100)
        def _():
            sl = pl.ds(cid * 100, 100)
            pltpu.sync_copy(acc_sh.at[sl, :], zbuf)
            pltpu.sync_copy(zbuf, acc_hbm.at[cc, sl, :])
        return carry
    lax.fori_loop(0, (N // 100 + NS - 1) // NS, ochunk, 0)


_edge = pl.kernel(
    _edge_body,
    out_type=jax.ShapeDtypeStruct((NC, N, ZW), f32),
    mesh=_MESH,
    compiler_params=_NOTC,
    scratch_types=[
        pltpu.VMEM((CH, ZW), f32),
        pltpu.VMEM((CH, 16), f32),
        pltpu.VMEM((CH, 16), f32),
        pltpu.VMEM((CH,), i32),
        pltpu.VMEM((CH,), i32),
        pltpu.VMEM((16,), f32),
        pltpu.VMEM((100, ZW), f32),
        pltpu.VMEM_SHARED((N, ZW), f32),
    ],
)


def _run(h, src2, dst2, W_emb, b_emb, W, a, gamma, beta,
         M0, mb0, M1, mb1, M2, mb2):
    L = W.shape[0]
    Wcat = jnp.transpose(W, (0, 2, 1, 3)).reshape(L, D, D)
    cidx = jnp.arange(D)
    hof = cidx // DH
    jof = cidx % DH
    a_s = a[:, hof, jof]
    a_d = a[:, hof, DH + jof]
    A_all = jnp.zeros((L, D, 32), f32)
    A_all = A_all.at[:, cidx, hof].set(a_s)
    A_all = A_all.at[:, cidx, 8 + hof].set(a_d)
    A_all = A_all.at[:, cidx, 16 + hof].set(a_d)
    P = (jnp.arange(8)[:, None] == hof[None, :]).astype(f32)
    gam = gamma.reshape(L, 1, D)
    bet = beta.reshape(L, 1, D)

    hcur, zr, sd, ds, sm = _dense0(h, W_emb, b_emb.reshape(1, D),
                                   Wcat[0], A_all[0])
    out = None
    for l in range(L):
        acc = _edge(sd, ds, sm.reshape(16), zr, src2, dst2)
        if l + 1 < L:
            hcur, zr, sd, ds, sm = _post(hcur, acc[0], acc[1], gam[l], bet[l],
                                         P, Wcat[l + 1], A_all[l + 1])
        else:
            out = _final(hcur, acc[0], acc[1], gam[l], bet[l], P,
                         M0, mb0.reshape(1, 64), M1, mb1.reshape(1, 32),
                         M2, mb2.reshape(1, 10))
    return out


def kernel(h, edge_index, e, W_emb, b_emb, W, a, gamma, beta,
           M0, mb0, M1, mb1, M2, mb2):
    out_dtype = jnp.result_type(h.dtype, W_emb.dtype, M2.dtype)
    h, W_emb, b_emb, W, a, gamma, beta = (
        x.astype(f32) for x in (h, W_emb, b_emb, W, a, gamma, beta))
    M0, mb0, M1, mb1, M2, mb2 = (
        x.astype(f32) for x in (M0, mb0, M1, mb1, M2, mb2))
    edge_index = edge_index.astype(i32)
    with jax.enable_x64(False):
        src2 = edge_index[0].reshape(NW, NCH, CH)
        dst2 = edge_index[1].reshape(NW, NCH, CH)
        out = _run(h, src2, dst2, W_emb, b_emb, W, a, gamma, beta,
                   M0, mb0, M1, mb1, M2, mb2)
    return out.astype(out_dtype)


# speedup vs baseline: 1597.1503x; 1.2357x over previous
import jax
import jax.numpy as jnp
from jax import lax
from jax.experimental import pallas as pl
from jax.experimental.pallas import tpu as pltpu
from jax.experimental.pallas import tpu_sc as plsc

N = 10000
E = 320000
H = 8
DH = 16
D = H * DH
NC = 2
NS = 16
NW = NC * NS
EW = E // NW
CH = 80
NCH = EW // CH
ZW = 144
RB = 1000
NEG = -1e30

f32 = jnp.float32
i32 = jnp.int32


def _prep_outputs(z, i, zr_ref, sd_ref, ds_ref, sm_ref, aa_ref):
    sd2 = jnp.dot(z, aa_ref[...], preferred_element_type=f32)
    sd_ref[...] = sd2[:, 0:16]
    ds_ref[...] = sd2[:, 16:32]
    rb = z.shape[0]
    zr_ref[...] = jnp.concatenate(
        [z, jnp.ones((rb, 8), f32), jnp.zeros((rb, 8), f32)], axis=1)
    bm = jnp.max(sd2[:, 0:16], axis=0, keepdims=True)

    @pl.when(i == 0)
    def _():
        sm_ref[...] = jnp.full((1, 16), NEG, f32)

    sm_ref[...] = jnp.maximum(sm_ref[...], bm)


def _dense0_body(h_ref, we_ref, be_ref, wc_ref, aa_ref,
                 h0_ref, zr_ref, sd_ref, ds_ref, sm_ref):
    i = pl.program_id(0)
    h0 = jnp.dot(h_ref[...], we_ref[...], preferred_element_type=f32) + be_ref[...]
    h0_ref[...] = h0
    z = jnp.dot(h0, wc_ref[...], preferred_element_type=f32)
    _prep_outputs(z, i, zr_ref, sd_ref, ds_ref, sm_ref, aa_ref)


def _post_head(hin_ref, a0_ref, a1_ref, g_ref, b_ref, p_ref):
    acc = a0_ref[...] + a1_ref[...]
    den = jnp.dot(acc[:, 128:136], p_ref[...], preferred_element_type=f32) + 1e-9
    t = acc[:, 0:128] / den * g_ref[...] + b_ref[...]
    hn = jnp.where(t > 0, t, jnp.exp(jnp.minimum(t, 0.0)) - 1.0)
    return hin_ref[...] + hn


def _post_body(hin_ref, a0_ref, a1_ref, g_ref, b_ref, p_ref, wc_ref, aa_ref,
               hn_ref, zr_ref, sd_ref, ds_ref, sm_ref):
    i = pl.program_id(0)
    hv = _post_head(hin_ref, a0_ref, a1_ref, g_ref, b_ref, p_ref)
    hn_ref[...] = hv
    z = jnp.dot(hv, wc_ref[...], preferred_element_type=f32)
    _prep_outputs(z, i, zr_ref, sd_ref, ds_ref, sm_ref, aa_ref)


def _final_body(hin_ref, a0_ref, a1_ref, g_ref, b_ref, p_ref,
                m0_ref, b0_ref, m1_ref, b1_ref, m2_ref, b2_ref,
                out_ref, sacc_ref):
    i = pl.program_id(0)
    hv = _post_head(hin_ref, a0_ref, a1_ref, g_ref, b_ref, p_ref)
    part = jnp.sum(hv, axis=0, keepdims=True)

    @pl.when(i == 0)
    def _():
        sacc_ref[...] = jnp.zeros_like(sacc_ref)

    sacc_ref[...] += part

    @pl.when(i == pl.num_programs(0) - 1)
    def _():
        hg = sacc_ref[...] * (1.0 / N)
        x = jnp.maximum(
            jnp.dot(hg, m0_ref[...], preferred_element_type=f32) + b0_ref[...], 0.0)
        x = jnp.maximum(
            jnp.dot(x, m1_ref[...], preferred_element_type=f32) + b1_ref[...], 0.0)
        out_ref[...] = jnp.dot(x, m2_ref[...], preferred_element_type=f32) + b2_ref[...]


def _row_spec(w):
    return pl.BlockSpec((RB, w), lambda i: (i, 0))


def _full_spec(r, c):
    return pl.BlockSpec((r, c), lambda i: (0, 0))


_GRID = N // RB

_STATE_SHAPES = [jax.ShapeDtypeStruct((N, D), f32),
                 jax.ShapeDtypeStruct((N, ZW), f32),
                 jax.ShapeDtypeStruct((N, 16), f32),
                 jax.ShapeDtypeStruct((N, 16), f32),
                 jax.ShapeDtypeStruct((1, 16), f32)]
_STATE_SPECS = [_row_spec(D), _row_spec(ZW), _row_spec(16), _row_spec(16),
                _full_spec(1, 16)]

_dense0 = pl.pallas_call(
    _dense0_body,
    grid=(_GRID,),
    in_specs=[_row_spec(D), _full_spec(D, D), _full_spec(1, D),
              _full_spec(D, D), _full_spec(D, 32)],
    out_specs=_STATE_SPECS,
    out_shape=_STATE_SHAPES,
)

_post = pl.pallas_call(
    _post_body,
    grid=(_GRID,),
    in_specs=[_row_spec(D), _row_spec(ZW), _row_spec(ZW), _full_spec(1, D),
              _full_spec(1, D), _full_spec(8, D), _full_spec(D, D),
              _full_spec(D, 32)],
    out_specs=_STATE_SPECS,
    out_shape=_STATE_SHAPES,
)

_final = pl.pallas_call(
    _final_body,
    grid=(_GRID,),
    in_specs=[_row_spec(D), _row_spec(ZW), _row_spec(ZW), _full_spec(1, D),
              _full_spec(1, D), _full_spec(8, D), _full_spec(D, 64),
              _full_spec(1, 64), _full_spec(64, 32), _full_spec(1, 32),
              _full_spec(32, 10), _full_spec(1, 10)],
    out_specs=pl.BlockSpec((1, 10), lambda i: (0, 0)),
    out_shape=jax.ShapeDtypeStruct((1, 10), f32),
    scratch_shapes=[pltpu.VMEM((1, D), f32)],
)


_MESH = plsc.VectorSubcoreMesh(core_axis_name="c", subcore_axis_name="s")
_NOTC = pltpu.CompilerParams(use_tc_tiling_on_sc=False)

_GDN = lax.GatherDimensionNumbers(
    offset_dims=(), collapsed_slice_dims=(0,), start_index_map=(0,))


def _vgather(vec, idx):
    return lax.gather(vec, idx[:, None], _GDN, (1,),
                      mode=lax.GatherScatterMode.PROMISE_IN_BOUNDS)


def _edge_body(sd_hbm, ds_hbm, sm_hbm, zr_hbm, src2_hbm, dst2_hbm,
               acc_hbm, zv, av, bv, srcv, dstv, smv, zbuf, acc_sh):
    cc = lax.axis_index("c")
    ss = lax.axis_index("s")
    wid = ss * NC + cc
    iota = lax.iota(i32, 16)
    lane8 = iota < 8
    zero = jnp.zeros((16,), f32)

    pltpu.sync_copy(sm_hbm, smv)

    def zrow(j, carry):
        for c9 in range(ZW // 16):
            zbuf[j, pl.ds(c9 * 16, 16)] = zero
        return carry
    lax.fori_loop(0, 100, zrow, 0)

    def zchunk(k, carry):
        cid = ss + k * NS

        @pl.when(cid < N // 100)
        def _():
            pltpu.sync_copy(zbuf, acc_sh.at[pl.ds(cid * 100, 100), :])
        return carry
    lax.fori_loop(0, (N // 100 + NS - 1) // NS, zchunk, 0)
    plsc.subcore_barrier()

    smaxv = smv[pl.ds(0, 16)]

    def chunk(ci, carry):
        pltpu.sync_copy(src2_hbm.at[wid, ci], srcv)
        pltpu.sync_copy(dst2_hbm.at[wid, ci], dstv)
        pltpu.sync_copy(sd_hbm.at[srcv], av)
        pltpu.sync_copy(ds_hbm.at[dstv], bv)
        pltpu.sync_copy(zr_hbm.at[srcv], zv)

        @plsc.parallel_loop(0, CH, 1, unroll=4)
        def edge(k):
            a = av[k]
            b = bv[k]
            t = a + b
            lr = jnp.maximum(t, t * 0.01)
            dg = b + smaxv
            m = jnp.maximum(dg, dg * 0.01)
            ex = jnp.exp(lr - m)
            exm = jnp.where(lane8, ex, 0.0)
            for hh in range(H):
                bc = _vgather(exm, jnp.full((16,), hh, i32))
                sl = pl.ds(hh * 16, 16)
                zv[k, sl] = zv[k, sl] * bc
            zv[k, pl.ds(128, 16)] = exm
        pltpu.sync_copy(zv, acc_sh.at[dstv], add=True)
        return carry
    lax.fori_loop(0, NCH, chunk, 0)

    plsc.subcore_barrier()

    def ochunk(k, carry):
        cid = ss + k * NS

        @pl.when(cid < N // 100)
        def _():
            sl = pl.ds(cid * 100, 100)
            pltpu.sync_copy(acc_sh.at[sl, :], zbuf)
            pltpu.sync_copy(zbuf, acc_hbm.at[cc, sl, :])
        return carry
    lax.fori_loop(0, (N // 100 + NS - 1) // NS, ochunk, 0)


_edge = pl.kernel(
    _edge_body,
    out_type=jax.ShapeDtypeStruct((NC, N, ZW), f32),
    mesh=_MESH,
    compiler_params=_NOTC,
    scratch_types=[
        pltpu.VMEM((CH, ZW), f32),
        pltpu.VMEM((CH, 16), f32),
        pltpu.VMEM((CH, 16), f32),
        pltpu.VMEM((CH,), i32),
        pltpu.VMEM((CH,), i32),
        pltpu.VMEM((16,), f32),
        pltpu.VMEM((100, ZW), f32),
        pltpu.VMEM_SHARED((N, ZW), f32),
    ],
)


def _run(h, src2, dst2, W_emb, b_emb, W, a, gamma, beta,
         M0, mb0, M1, mb1, M2, mb2):
    L = W.shape[0]
    Wcat = jnp.transpose(W, (0, 2, 1, 3)).reshape(L, D, D)
    cidx = jnp.arange(D)
    hof = cidx // DH
    jof = cidx % DH
    a_s = a[:, hof, jof]
    a_d = a[:, hof, DH + jof]
    A_all = jnp.zeros((L, D, 32), f32)
    A_all = A_all.at[:, cidx, hof].set(a_s)
    A_all = A_all.at[:, cidx, 8 + hof].set(a_d)
    A_all = A_all.at[:, cidx, 16 + hof].set(a_d)
    P = (jnp.arange(8)[:, None] == hof[None, :]).astype(f32)
    gam = gamma.reshape(L, 1, D)
    bet = beta.reshape(L, 1, D)

    hcur, zr, sd, ds, sm = _dense0(h, W_emb, b_emb.reshape(1, D),
                                   Wcat[0], A_all[0])
    out = None
    for l in range(L):
        acc = _edge(sd, ds, sm.reshape(16), zr, src2, dst2)
        if l + 1 < L:
            hcur, zr, sd, ds, sm = _post(hcur, acc[0], acc[1], gam[l], bet[l],
                                         P, Wcat[l + 1], A_all[l + 1])
        else:
            out = _final(hcur, acc[0], acc[1], gam[l], bet[l], P,
                         M0, mb0.reshape(1, 64), M1, mb1.reshape(1, 32),
                         M2, mb2.reshape(1, 10))
    return out


def kernel(h, edge_index, e, W_emb, b_emb, W, a, gamma, beta,
           M0, mb0, M1, mb1, M2, mb2):
    out_dtype = jnp.result_type(h.dtype, W_emb.dtype, M2.dtype)
    h, W_emb, b_emb, W, a, gamma, beta = (
        x.astype(f32) for x in (h, W_emb, b_emb, W, a, gamma, beta))
    M0, mb0, M1, mb1, M2, mb2 = (
        x.astype(f32) for x in (M0, mb0, M1, mb1, M2, mb2))
    edge_index = edge_index.astype(i32)
    with jax.enable_x64(False):
        src2 = edge_index[0].reshape(NW, NCH, CH)
        dst2 = edge_index[1].reshape(NW, NCH, CH)
        out = _run(h, src2, dst2, W_emb, b_emb, W, a, gamma, beta,
                   M0, mb0, M1, mb1, M2, mb2)
    return out.astype(out_dtype)


# speedup vs baseline: 1603.0127x; 1.0037x over previous
import jax
import jax.numpy as jnp
from jax import lax
from jax.experimental import pallas as pl
from jax.experimental.pallas import tpu as pltpu
from jax.experimental.pallas import tpu_sc as plsc

N = 10000
E = 320000
H = 8
DH = 16
D = H * DH
NC = 2
NS = 16
NW = NC * NS
EW = E // NW
CH = 80
NCH = EW // CH
ZW = 144
RB = 1000
NEG = -1e30

f32 = jnp.float32
i32 = jnp.int32


def _prep_outputs(z, i, zr_ref, sd_ref, ds_ref, sm_ref, aa_ref):
    sd2 = jnp.dot(z, aa_ref[...], preferred_element_type=f32)
    sd_ref[...] = sd2[:, 0:16]
    ds_ref[...] = sd2[:, 16:32]
    rb = z.shape[0]
    zr_ref[...] = jnp.concatenate(
        [z, jnp.ones((rb, 8), f32), jnp.zeros((rb, 8), f32)], axis=1)
    bm = jnp.max(sd2[:, 0:16], axis=0, keepdims=True)

    @pl.when(i == 0)
    def _():
        sm_ref[...] = jnp.full((1, 16), NEG, f32)

    sm_ref[...] = jnp.maximum(sm_ref[...], bm)


def _dense0_body(h_ref, we_ref, be_ref, wc_ref, aa_ref,
                 h0_ref, zr_ref, sd_ref, ds_ref, sm_ref):
    i = pl.program_id(0)
    h0 = jnp.dot(h_ref[...], we_ref[...], preferred_element_type=f32) + be_ref[...]
    h0_ref[...] = h0
    z = jnp.dot(h0, wc_ref[...], preferred_element_type=f32)
    _prep_outputs(z, i, zr_ref, sd_ref, ds_ref, sm_ref, aa_ref)


def _post_head(hin_ref, a0_ref, a1_ref, g_ref, b_ref, p_ref):
    acc = a0_ref[...] + a1_ref[...]
    den = jnp.dot(acc[:, 128:136], p_ref[...], preferred_element_type=f32) + 1e-9
    t = acc[:, 0:128] / den * g_ref[...] + b_ref[...]
    hn = jnp.where(t > 0, t, jnp.exp(jnp.minimum(t, 0.0)) - 1.0)
    return hin_ref[...] + hn


def _post_body(hin_ref, a0_ref, a1_ref, g_ref, b_ref, p_ref, wc_ref, aa_ref,
               hn_ref, zr_ref, sd_ref, ds_ref, sm_ref):
    i = pl.program_id(0)
    hv = _post_head(hin_ref, a0_ref, a1_ref, g_ref, b_ref, p_ref)
    hn_ref[...] = hv
    z = jnp.dot(hv, wc_ref[...], preferred_element_type=f32)
    _prep_outputs(z, i, zr_ref, sd_ref, ds_ref, sm_ref, aa_ref)


def _final_body(hin_ref, a0_ref, a1_ref, g_ref, b_ref, p_ref,
                m0_ref, b0_ref, m1_ref, b1_ref, m2_ref, b2_ref,
                out_ref, sacc_ref):
    i = pl.program_id(0)
    hv = _post_head(hin_ref, a0_ref, a1_ref, g_ref, b_ref, p_ref)
    part = jnp.sum(hv, axis=0, keepdims=True)

    @pl.when(i == 0)
    def _():
        sacc_ref[...] = jnp.zeros_like(sacc_ref)

    sacc_ref[...] += part

    @pl.when(i == pl.num_programs(0) - 1)
    def _():
        hg = sacc_ref[...] * (1.0 / N)
        x = jnp.maximum(
            jnp.dot(hg, m0_ref[...], preferred_element_type=f32) + b0_ref[...], 0.0)
        x = jnp.maximum(
            jnp.dot(x, m1_ref[...], preferred_element_type=f32) + b1_ref[...], 0.0)
        out_ref[...] = jnp.dot(x, m2_ref[...], preferred_element_type=f32) + b2_ref[...]


def _row_spec(w):
    return pl.BlockSpec((RB, w), lambda i: (i, 0))


def _full_spec(r, c):
    return pl.BlockSpec((r, c), lambda i: (0, 0))


_GRID = N // RB

_STATE_SHAPES = [jax.ShapeDtypeStruct((N, D), f32),
                 jax.ShapeDtypeStruct((N, ZW), f32),
                 jax.ShapeDtypeStruct((N, 16), f32),
                 jax.ShapeDtypeStruct((N, 16), f32),
                 jax.ShapeDtypeStruct((1, 16), f32)]
_STATE_SPECS = [_row_spec(D), _row_spec(ZW), _row_spec(16), _row_spec(16),
                _full_spec(1, 16)]

_dense0 = pl.pallas_call(
    _dense0_body,
    grid=(_GRID,),
    in_specs=[_row_spec(D), _full_spec(D, D), _full_spec(1, D),
              _full_spec(D, D), _full_spec(D, 32)],
    out_specs=_STATE_SPECS,
    out_shape=_STATE_SHAPES,
)

_post = pl.pallas_call(
    _post_body,
    grid=(_GRID,),
    in_specs=[_row_spec(D), _row_spec(ZW), _row_spec(ZW), _full_spec(1, D),
              _full_spec(1, D), _full_spec(8, D), _full_spec(D, D),
              _full_spec(D, 32)],
    out_specs=_STATE_SPECS,
    out_shape=_STATE_SHAPES,
)

_final = pl.pallas_call(
    _final_body,
    grid=(_GRID,),
    in_specs=[_row_spec(D), _row_spec(ZW), _row_spec(ZW), _full_spec(1, D),
              _full_spec(1, D), _full_spec(8, D), _full_spec(D, 64),
              _full_spec(1, 64), _full_spec(64, 32), _full_spec(1, 32),
              _full_spec(32, 10), _full_spec(1, 10)],
    out_specs=pl.BlockSpec((1, 10), lambda i: (0, 0)),
    out_shape=jax.ShapeDtypeStruct((1, 10), f32),
    scratch_shapes=[pltpu.VMEM((1, D), f32)],
)


_MESH = plsc.VectorSubcoreMesh(core_axis_name="c", subcore_axis_name="s")
_NOTC = pltpu.CompilerParams(use_tc_tiling_on_sc=False)

_GDN = lax.GatherDimensionNumbers(
    offset_dims=(), collapsed_slice_dims=(0,), start_index_map=(0,))


def _vgather(vec, idx):
    return lax.gather(vec, idx[:, None], _GDN, (1,),
                      mode=lax.GatherScatterMode.PROMISE_IN_BOUNDS)


def _edge_body(sd_hbm, ds_hbm, sm_hbm, zr_hbm, src2_hbm, dst2_hbm,
               acc_hbm, zv, av, bv, srcv, dstv, smv, zbuf, acc_sh):
    cc = lax.axis_index("c")
    ss = lax.axis_index("s")
    wid = ss * NC + cc
    iota = lax.iota(i32, 16)
    lane8 = iota < 8
    idx8 = jnp.where(lane8, iota, iota - 8)
    zero = jnp.zeros((16,), f32)

    pltpu.sync_copy(sm_hbm, smv)

    def zrow(j, carry):
        for c9 in range(ZW // 16):
            zbuf[j, pl.ds(c9 * 16, 16)] = zero
        return carry
    lax.fori_loop(0, 100, zrow, 0)

    def zchunk(k, carry):
        cid = ss + k * NS

        @pl.when(cid < N // 100)
        def _():
            pltpu.sync_copy(zbuf, acc_sh.at[pl.ds(cid * 100, 100), :])
        return carry
    lax.fori_loop(0, (N // 100 + NS - 1) // NS, zchunk, 0)
    plsc.subcore_barrier()

    smaxv = smv[pl.ds(0, 16)]

    def chunk(ci, carry):
        pltpu.sync_copy(src2_hbm.at[wid, ci], srcv)
        pltpu.sync_copy(dst2_hbm.at[wid, ci], dstv)
        pltpu.sync_copy(sd_hbm.at[srcv], av)
        pltpu.sync_copy(ds_hbm.at[dstv], bv)
        pltpu.sync_copy(zr_hbm.at[srcv], zv)

        @plsc.parallel_loop(0, CH, 1, unroll=4)
        def edge(k):
            a = av[k]
            b = bv[k]
            t = a + b
            lr = jnp.maximum(t, t * 0.01)
            dg = b + smaxv
            m = jnp.maximum(dg, dg * 0.01)
            ex = jnp.exp(lr - m)
            exm = jnp.where(lane8, ex, 0.0)
            exd = _vgather(exm, idx8)
            for c8 in range(8):
                sl = pl.ds(c8 * 16, 16)
                zv[k, sl] = zv[k, sl] * exd
            zv[k, pl.ds(128, 16)] = exm
        pltpu.sync_copy(zv, acc_sh.at[dstv], add=True)
        return carry
    lax.fori_loop(0, NCH, chunk, 0)

    plsc.subcore_barrier()

    def ochunk(k, carry):
        cid = ss + k * NS

        @pl.when(cid < N // 100)
        def _():
            sl = pl.ds(cid * 100, 100)
            pltpu.sync_copy(acc_sh.at[sl, :], zbuf)
            pltpu.sync_copy(zbuf, acc_hbm.at[cc, sl, :])
        return carry
    lax.fori_loop(0, (N // 100 + NS - 1) // NS, ochunk, 0)


_edge = pl.kernel(
    _edge_body,
    out_type=jax.ShapeDtypeStruct((NC, N, ZW), f32),
    mesh=_MESH,
    compiler_params=_NOTC,
    scratch_types=[
        pltpu.VMEM((CH, ZW), f32),
        pltpu.VMEM((CH, 16), f32),
        pltpu.VMEM((CH, 16), f32),
        pltpu.VMEM((CH,), i32),
        pltpu.VMEM((CH,), i32),
        pltpu.VMEM((16,), f32),
        pltpu.VMEM((100, ZW), f32),
        pltpu.VMEM_SHARED((N, ZW), f32),
    ],
)


def _run(h, src2, dst2, W_emb, b_emb, W, a, gamma, beta,
         M0, mb0, M1, mb1, M2, mb2):
    L = W.shape[0]
    cidx = jnp.arange(D)
    hof = cidx % 8
    jof = cidx // 8
    PERM = hof * DH + jof
    Wcat = jnp.transpose(W, (0, 2, 1, 3)).reshape(L, D, D)[:, PERM][:, :, PERM]
    a_s = a[:, hof, jof]
    a_d = a[:, hof, DH + jof]
    A_all = jnp.zeros((L, D, 32), f32)
    A_all = A_all.at[:, cidx, hof].set(a_s)
    A_all = A_all.at[:, cidx, 8 + hof].set(a_d)
    A_all = A_all.at[:, cidx, 16 + hof].set(a_d)
    P = (jnp.arange(8)[:, None] == hof[None, :]).astype(f32)
    gam = jnp.transpose(gamma, (0, 2, 1)).reshape(L, 1, D)
    bet = jnp.transpose(beta, (0, 2, 1)).reshape(L, 1, D)
    W_emb = W_emb[:, PERM]
    b_emb = b_emb[PERM]
    M0 = M0[PERM]

    hcur, zr, sd, ds, sm = _dense0(h, W_emb, b_emb.reshape(1, D),
                                   Wcat[0], A_all[0])
    out = None
    for l in range(L):
        acc = _edge(sd, ds, sm.reshape(16), zr, src2, dst2)
        if l + 1 < L:
            hcur, zr, sd, ds, sm = _post(hcur, acc[0], acc[1], gam[l], bet[l],
                                         P, Wcat[l + 1], A_all[l + 1])
        else:
            out = _final(hcur, acc[0], acc[1], gam[l], bet[l], P,
                         M0, mb0.reshape(1, 64), M1, mb1.reshape(1, 32),
                         M2, mb2.reshape(1, 10))
    return out


def kernel(h, edge_index, e, W_emb, b_emb, W, a, gamma, beta,
           M0, mb0, M1, mb1, M2, mb2):
    out_dtype = jnp.result_type(h.dtype, W_emb.dtype, M2.dtype)
    h, W_emb, b_emb, W, a, gamma, beta = (
        x.astype(f32) for x in (h, W_emb, b_emb, W, a, gamma, beta))
    M0, mb0, M1, mb1, M2, mb2 = (
        x.astype(f32) for x in (M0, mb0, M1, mb1, M2, mb2))
    edge_index = edge_index.astype(i32)
    with jax.enable_x64(False):
        src2 = edge_index[0].reshape(NW, NCH, CH)
        dst2 = edge_index[1].reshape(NW, NCH, CH)
        out = _run(h, src2, dst2, W_emb, b_emb, W, a, gamma, beta,
                   M0, mb0, M1, mb1, M2, mb2)
    return out.astype(out_dtype)


# speedup vs baseline: 1603.4685x; 1.0003x over previous
import jax
import jax.numpy as jnp
from jax import lax
from jax.experimental import pallas as pl
from jax.experimental.pallas import tpu as pltpu
from jax.experimental.pallas import tpu_sc as plsc

N = 10000
E = 320000
H = 8
DH = 16
D = H * DH
NC = 2
NS = 16
NW = NC * NS
EW = E // NW
CH = 80
NCH = EW // CH
ZW = 144
RB = 1000
NEG = -1e30

f32 = jnp.float32
i32 = jnp.int32


def _prep_outputs(z, i, zr_ref, sd_ref, ds_ref, sm_ref, aa_ref):
    sd2 = jnp.dot(z, aa_ref[...], preferred_element_type=f32)
    sd_ref[...] = sd2[:, 0:16]
    ds_ref[...] = sd2[:, 16:32]
    rb = z.shape[0]
    zr_ref[...] = jnp.concatenate(
        [z, jnp.ones((rb, 8), f32), jnp.zeros((rb, 8), f32)], axis=1)
    bm = jnp.max(sd2[:, 0:16], axis=0, keepdims=True)

    @pl.when(i == 0)
    def _():
        sm_ref[...] = jnp.full((1, 16), NEG, f32)

    sm_ref[...] = jnp.maximum(sm_ref[...], bm)


def _dense0_body(h_ref, we_ref, be_ref, wc_ref, aa_ref,
                 h0_ref, zr_ref, sd_ref, ds_ref, sm_ref):
    i = pl.program_id(0)
    h0 = jnp.dot(h_ref[...], we_ref[...], preferred_element_type=f32) + be_ref[...]
    h0_ref[...] = h0
    z = jnp.dot(h0, wc_ref[...], preferred_element_type=f32)
    _prep_outputs(z, i, zr_ref, sd_ref, ds_ref, sm_ref, aa_ref)


def _post_head(hin_ref, a0_ref, a1_ref, g_ref, b_ref, p_ref):
    acc = a0_ref[...] + a1_ref[...]
    den = jnp.dot(acc[:, 128:136], p_ref[...], preferred_element_type=f32)
    ratio = jnp.where(den > 0, acc[:, 0:128] / den, 0.0)
    t = ratio * g_ref[...] + b_ref[...]
    hn = jnp.where(t > 0, t, jnp.exp(jnp.minimum(t, 0.0)) - 1.0)
    return hin_ref[...] + hn


def _post_body(hin_ref, a0_ref, a1_ref, g_ref, b_ref, p_ref, wc_ref, aa_ref,
               hn_ref, zr_ref, sd_ref, ds_ref, sm_ref):
    i = pl.program_id(0)
    hv = _post_head(hin_ref, a0_ref, a1_ref, g_ref, b_ref, p_ref)
    hn_ref[...] = hv
    z = jnp.dot(hv, wc_ref[...], preferred_element_type=f32)
    _prep_outputs(z, i, zr_ref, sd_ref, ds_ref, sm_ref, aa_ref)


def _final_body(hin_ref, a0_ref, a1_ref, g_ref, b_ref, p_ref,
                m0_ref, b0_ref, m1_ref, b1_ref, m2_ref, b2_ref,
                out_ref, sacc_ref):
    i = pl.program_id(0)
    hv = _post_head(hin_ref, a0_ref, a1_ref, g_ref, b_ref, p_ref)
    part = jnp.sum(hv, axis=0, keepdims=True)

    @pl.when(i == 0)
    def _():
        sacc_ref[...] = jnp.zeros_like(sacc_ref)

    sacc_ref[...] += part

    @pl.when(i == pl.num_programs(0) - 1)
    def _():
        hg = sacc_ref[...] * (1.0 / N)
        x = jnp.maximum(
            jnp.dot(hg, m0_ref[...], preferred_element_type=f32) + b0_ref[...], 0.0)
        x = jnp.maximum(
            jnp.dot(x, m1_ref[...], preferred_element_type=f32) + b1_ref[...], 0.0)
        out_ref[...] = jnp.dot(x, m2_ref[...], preferred_element_type=f32) + b2_ref[...]


def _row_spec(w):
    return pl.BlockSpec((RB, w), lambda i: (i, 0))


def _full_spec(r, c):
    return pl.BlockSpec((r, c), lambda i: (0, 0))


_GRID = N // RB

_STATE_SHAPES = [jax.ShapeDtypeStruct((N, D), f32),
                 jax.ShapeDtypeStruct((N, ZW), f32),
                 jax.ShapeDtypeStruct((N, 16), f32),
                 jax.ShapeDtypeStruct((N, 16), f32),
                 jax.ShapeDtypeStruct((1, 16), f32)]
_STATE_SPECS = [_row_spec(D), _row_spec(ZW), _row_spec(16), _row_spec(16),
                _full_spec(1, 16)]

_dense0 = pl.pallas_call(
    _dense0_body,
    grid=(_GRID,),
    in_specs=[_row_spec(D), _full_spec(D, D), _full_spec(1, D),
              _full_spec(D, D), _full_spec(D, 32)],
    out_specs=_STATE_SPECS,
    out_shape=_STATE_SHAPES,
)

_post = pl.pallas_call(
    _post_body,
    grid=(_GRID,),
    in_specs=[_row_spec(D), _row_spec(ZW), _row_spec(ZW), _full_spec(1, D),
              _full_spec(1, D), _full_spec(8, D), _full_spec(D, D),
              _full_spec(D, 32)],
    out_specs=_STATE_SPECS,
    out_shape=_STATE_SHAPES,
)

_final = pl.pallas_call(
    _final_body,
    grid=(_GRID,),
    in_specs=[_row_spec(D), _row_spec(ZW), _row_spec(ZW), _full_spec(1, D),
              _full_spec(1, D), _full_spec(8, D), _full_spec(D, 64),
              _full_spec(1, 64), _full_spec(64, 32), _full_spec(1, 32),
              _full_spec(32, 10), _full_spec(1, 10)],
    out_specs=pl.BlockSpec((1, 10), lambda i: (0, 0)),
    out_shape=jax.ShapeDtypeStruct((1, 10), f32),
    scratch_shapes=[pltpu.VMEM((1, D), f32)],
)


_MESH = plsc.VectorSubcoreMesh(core_axis_name="c", subcore_axis_name="s")
_NOTC = pltpu.CompilerParams(use_tc_tiling_on_sc=False)

_GDN = lax.GatherDimensionNumbers(
    offset_dims=(), collapsed_slice_dims=(0,), start_index_map=(0,))


def _vgather(vec, idx):
    return lax.gather(vec, idx[:, None], _GDN, (1,),
                      mode=lax.GatherScatterMode.PROMISE_IN_BOUNDS)


def _edge_body(sd_hbm, ds_hbm, sm_hbm, zr_hbm, src2_hbm, dst2_hbm,
               acc_hbm, zv, av, bv, srcv, dstv, smv, zbuf, acc_sh):
    cc = lax.axis_index("c")
    ss = lax.axis_index("s")
    wid = ss * NC + cc
    iota = lax.iota(i32, 16)
    lane8 = iota < 8
    idx8 = jnp.where(lane8, iota, iota - 8)
    zero = jnp.zeros((16,), f32)

    pltpu.sync_copy(sm_hbm, smv)

    def zrow(j, carry):
        for c9 in range(ZW // 16):
            zbuf[j, pl.ds(c9 * 16, 16)] = zero
        return carry
    lax.fori_loop(0, 100, zrow, 0)

    def zchunk(k, carry):
        cid = ss + k * NS

        @pl.when(cid < N // 100)
        def _():
            pltpu.sync_copy(zbuf, acc_sh.at[pl.ds(cid * 100, 100), :])
        return carry
    lax.fori_loop(0, (N // 100 + NS - 1) // NS, zchunk, 0)
    plsc.subcore_barrier()

    smaxv = smv[pl.ds(0, 16)]

    def chunk(ci, carry):
        pltpu.sync_copy(src2_hbm.at[wid, ci], srcv)
        pltpu.sync_copy(dst2_hbm.at[wid, ci], dstv)
        pltpu.sync_copy(sd_hbm.at[srcv], av)
        pltpu.sync_copy(ds_hbm.at[dstv], bv)
        pltpu.sync_copy(zr_hbm.at[srcv], zv)

        @plsc.parallel_loop(0, CH, 1, unroll=4)
        def edge(k):
            a = av[k]
            b = bv[k]
            t = a + b
            lr = jnp.maximum(t, t * 0.01)
            dg = b + smaxv
            m = jnp.maximum(dg, dg * 0.01)
            ex = jnp.exp(lr - m)
            exm = jnp.where(lane8, ex, 0.0)
            exd = _vgather(exm, idx8)
            for c8 in range(8):
                sl = pl.ds(c8 * 16, 16)
                zv[k, sl] = zv[k, sl] * exd
            zv[k, pl.ds(128, 16)] = exm
        pltpu.sync_copy(zv, acc_sh.at[dstv], add=True)
        return carry
    lax.fori_loop(0, NCH, chunk, 0)

    plsc.subcore_barrier()

    def ochunk(k, carry):
        cid = ss + k * NS

        @pl.when(cid < N // 100)
        def _():
            sl = pl.ds(cid * 100, 100)
            pltpu.sync_copy(acc_sh.at[sl, :], zbuf)
            pltpu.sync_copy(zbuf, acc_hbm.at[cc, sl, :])
        return carry
    lax.fori_loop(0, (N // 100 + NS - 1) // NS, ochunk, 0)


_edge = pl.kernel(
    _edge_body,
    out_type=jax.ShapeDtypeStruct((NC, N, ZW), f32),
    mesh=_MESH,
    compiler_params=_NOTC,
    scratch_types=[
        pltpu.VMEM((CH, ZW), f32),
        pltpu.VMEM((CH, 16), f32),
        pltpu.VMEM((CH, 16), f32),
        pltpu.VMEM((CH,), i32),
        pltpu.VMEM((CH,), i32),
        pltpu.VMEM((16,), f32),
        pltpu.VMEM((100, ZW), f32),
        pltpu.VMEM_SHARED((N, ZW), f32),
    ],
)


def _run(h, src2, dst2, W_emb, b_emb, W, a, gamma, beta,
         M0, mb0, M1, mb1, M2, mb2):
    L = W.shape[0]
    cidx = jnp.arange(D)
    hof = cidx % 8
    jof = cidx // 8
    PERM = hof * DH + jof
    Wcat = jnp.transpose(W, (0, 2, 1, 3)).reshape(L, D, D)[:, PERM][:, :, PERM]
    a_s = a[:, hof, jof]
    a_d = a[:, hof, DH + jof]
    A_all = jnp.zeros((L, D, 32), f32)
    A_all = A_all.at[:, cidx, hof].set(a_s)
    A_all = A_all.at[:, cidx, 8 + hof].set(a_d)
    A_all = A_all.at[:, cidx, 16 + hof].set(a_d)
    P = (jnp.arange(8)[:, None] == hof[None, :]).astype(f32)
    gam = jnp.transpose(gamma, (0, 2, 1)).reshape(L, 1, D)
    bet = jnp.transpose(beta, (0, 2, 1)).reshape(L, 1, D)
    W_emb = W_emb[:, PERM]
    b_emb = b_emb[PERM]
    M0 = M0[PERM]

    hcur, zr, sd, ds, sm = _dense0(h, W_emb, b_emb.reshape(1, D),
                                   Wcat[0], A_all[0])
    out = None
    for l in range(L):
        acc = _edge(sd, ds, sm.reshape(16), zr, src2, dst2)
        if l + 1 < L:
            hcur, zr, sd, ds, sm = _post(hcur, acc[0], acc[1], gam[l], bet[l],
                                         P, Wcat[l + 1], A_all[l + 1])
        else:
            out = _final(hcur, acc[0], acc[1], gam[l], bet[l], P,
                         M0, mb0.reshape(1, 64), M1, mb1.reshape(1, 32),
                         M2, mb2.reshape(1, 10))
    return out


def kernel(h, edge_index, e, W_emb, b_emb, W, a, gamma, beta,
           M0, mb0, M1, mb1, M2, mb2):
    out_dtype = jnp.result_type(h.dtype, W_emb.dtype, M2.dtype)
    h, W_emb, b_emb, W, a, gamma, beta = (
        x.astype(f32) for x in (h, W_emb, b_emb, W, a, gamma, beta))
    M0, mb0, M1, mb1, M2, mb2 = (
        x.astype(f32) for x in (M0, mb0, M1, mb1, M2, mb2))
    edge_index = edge_index.astype(i32)
    with jax.enable_x64(False):
        src2 = edge_index[0].reshape(NW, NCH, CH)
        dst2 = edge_index[1].reshape(NW, NCH, CH)
        out = _run(h, src2, dst2, W_emb, b_emb, W, a, gamma, beta,
                   M0, mb0, M1, mb1, M2, mb2)
    return out.astype(out_dtype)
